# baseline (speedup 1.0000x reference)
import functools

import jax
import jax.numpy as jnp
from jax import lax
from jax.experimental import pallas as pl
from jax.experimental.pallas import tpu as pltpu
from jax.experimental.pallas import tpu_sc as plsc

N = 10000
E = 160000
H = 512
G = 64
L = 3

NC = 2
NS = 16
NTILES = NC * NS

EPT = E // NS
XSL = 4
HSL = H // XSL
NPAD = 10240
RPT = NPAD // NS
KB = 80
NBATCH = EPT // KB

DEA = 128
EPT2 = E // NTILES
KB2 = 40
NB2 = EPT2 // KB2

MB = 1000
NBLK = N // MB

_mesh = plsc.VectorSubcoreMesh(core_axis_name="c", subcore_axis_name="s")


def _zero_vmem_2d(ref, rows, cols):
    z = jnp.zeros((16,), jnp.float32)
    for r in range(rows):
        def body(j, carry):
            ref[r, pl.ds(j * 16, 16)] = z
            return carry
        lax.fori_loop(0, cols // 16, body, 0)


@functools.partial(
    pl.kernel,
    out_type=jax.ShapeDtypeStruct((NC, NPAD, DEA), jnp.float32),
    mesh=_mesh,
    scratch_types=[
        pltpu.VMEM_SHARED((NPAD, DEA), jnp.float32),
        pltpu.VMEM((KB2,), jnp.int32),
        pltpu.VMEM((KB2,), jnp.int32),
        pltpu.VMEM((KB2, DEA), jnp.float32),
        pltpu.VMEM((KB2,), jnp.int32),
        pltpu.VMEM((KB2,), jnp.int32),
        pltpu.VMEM((KB2, DEA), jnp.float32),
        pltpu.VMEM((16, DEA), jnp.float32),
        pltpu.SemaphoreType.DMA,
        pltpu.SemaphoreType.DMA,
    ],
)
def _sc_edge_feat(ea_hbm, row_hbm, col_hbm, p_hbm,
                  acc, st_r0, st_c0, ea_buf0, st_r1, st_c1, ea_buf1, zbuf,
                  sem0, sem1):
    ci = lax.axis_index("c")
    si = lax.axis_index("s")
    w = ci * NS + si
    bufs = ((st_r0, st_c0, ea_buf0, sem0), (st_r1, st_c1, ea_buf1, sem1))
    _zero_vmem_2d(zbuf, 16, DEA)

    def zero_acc(k, carry):
        pltpu.sync_copy(zbuf, acc.at[pl.ds(si * RPT + k * 16, 16)])
        return carry
    lax.fori_loop(0, RPT // 16, zero_acc, 0)
    plsc.subcore_barrier()

    def srcs(j):
        e0 = w * EPT2 + j * KB2
        return (row_hbm.at[pl.ds(e0, KB2)], col_hbm.at[pl.ds(e0, KB2)],
                ea_hbm.at[pl.ds(e0, KB2)])

    def start(j, bset):
        sr, sc, se = srcs(j)
        pltpu.async_copy(sr, bset[0], bset[3])
        pltpu.async_copy(sc, bset[1], bset[3])
        pltpu.async_copy(se, bset[2], bset[3])

    def wait(j, bset):
        sr, sc, se = srcs(j)
        pltpu.make_async_copy(sr, bset[0], bset[3]).wait()
        pltpu.make_async_copy(sc, bset[1], bset[3]).wait()
        pltpu.make_async_copy(se, bset[2], bset[3]).wait()

    def scat(bset):
        pltpu.sync_copy(bset[2], acc.at[bset[0]], add=True)
        pltpu.sync_copy(bset[2], acc.at[bset[1]], add=True)

    start(0, bufs[0])

    def pair(g, carry):
        b0 = 2 * g
        start(b0 + 1, bufs[1])
        wait(b0, bufs[0])
        scat(bufs[0])
        start(b0 + 2, bufs[0])
        wait(b0 + 1, bufs[1])
        scat(bufs[1])
        return carry
    lax.fori_loop(0, (NB2 - 1) // 2, pair, 0)
    wait(NB2 - 1, bufs[0])
    scat(bufs[0])

    plsc.subcore_barrier()
    pltpu.sync_copy(acc.at[pl.ds(si * RPT, RPT)],
                    p_hbm.at[ci, pl.ds(si * RPT, RPT)])


@functools.partial(
    pl.kernel,
    out_type=jax.ShapeDtypeStruct((XSL, NPAD, HSL), jnp.float32),
    mesh=_mesh,
    scratch_types=[
        pltpu.VMEM_SHARED((NPAD, HSL), jnp.float32),
        pltpu.VMEM((EPT,), jnp.int32),
        pltpu.VMEM((EPT,), jnp.int32),
        pltpu.VMEM((KB,), jnp.int32),
        pltpu.VMEM((KB, HSL), jnp.float32),
        pltpu.VMEM((KB, HSL), jnp.float32),
        pltpu.VMEM((16, HSL), jnp.float32),
        pltpu.SemaphoreType.DMA,
        pltpu.SemaphoreType.DMA,
    ],
)
def _sc_agg(xe_hbm, row_hbm, col_hbm, agg_hbm,
            acc, row_v, col_v, stage, rows_buf0, rows_buf1, zbuf, sem0, sem1):
    ci = lax.axis_index("c")
    si = lax.axis_index("s")
    ebase = si * EPT
    pltpu.sync_copy(row_hbm.at[pl.ds(ebase, EPT)], row_v)
    pltpu.sync_copy(col_hbm.at[pl.ds(ebase, EPT)], col_v)
    _zero_vmem_2d(zbuf, 16, HSL)
    bufs = ((rows_buf0, sem0), (rows_buf1, sem1))

    for sl in range(XSL // NC):
        slidx = ci * (XSL // NC) + sl
        def zero_acc(k, carry):
            pltpu.sync_copy(zbuf, acc.at[pl.ds(si * RPT + k * 16, 16)])
            return carry
        lax.fori_loop(0, RPT // 16, zero_acc, 0)
        plsc.subcore_barrier()

        def gsrc(b):
            return xe_hbm.at[slidx].at[row_v.at[pl.ds(b * KB, KB)]]

        def start(b, bset):
            pltpu.async_copy(gsrc(b), bset[0], bset[1])

        def wait(b, bset):
            pltpu.make_async_copy(gsrc(b), bset[0], bset[1]).wait()

        def scat(b, bset):
            for k in range(KB // 16):
                stage[pl.ds(k * 16, 16)] = col_v[pl.ds(b * KB + k * 16, 16)]
            pltpu.sync_copy(bset[0], acc.at[stage], add=True)

        start(0, bufs[0])

        def pair(g, carry):
            b0 = 2 * g
            start(b0 + 1, bufs[1])
            wait(b0, bufs[0])
            scat(b0, bufs[0])
            start(b0 + 2, bufs[0])
            wait(b0 + 1, bufs[1])
            scat(b0 + 1, bufs[1])
            return carry
        lax.fori_loop(0, (NBATCH - 1) // 2, pair, 0)
        wait(NBATCH - 1, bufs[0])
        scat(NBATCH - 1, bufs[0])
        plsc.subcore_barrier()

        pltpu.sync_copy(acc.at[pl.ds(si * RPT, RPT)],
                        agg_hbm.at[slidx, pl.ds(si * RPT, RPT)])
        plsc.subcore_barrier()


def _cat(ref3):
    return jnp.concatenate([ref3[k] for k in range(XSL)], axis=1)


def _split_store(ref3, val):
    for k in range(XSL):
        ref3[k] = val[:, k * HSL:(k + 1) * HSL]


def _pre_body(x_ref, nw_ref, nb_ref, p_ref, wa_ref, xe_ref, ec_ref):
    h = jnp.maximum(
        jnp.dot(x_ref[...], nw_ref[...], preferred_element_type=jnp.float32)
        + nb_ref[...], 0.0)
    ec = jnp.dot(p_ref[0] + p_ref[1], wa_ref[...],
                 preferred_element_type=jnp.float32)
    ec_ref[...] = ec
    _split_store(xe_ref, h + ec)


def _ka_body(s_ref, xe_ref, agg_ref, w1_ref, b1_ref, z1_ref, st_ref, acc_ref):
    i = pl.program_id(0)
    @pl.when(i == 0)
    def _():
        acc_ref[...] = jnp.zeros_like(acc_ref)
    zpre = s_ref[0, 0] * _cat(xe_ref) + _cat(agg_ref)
    z1 = jnp.dot(zpre.astype(jnp.bfloat16), w1_ref[...].astype(jnp.bfloat16),
                 preferred_element_type=jnp.float32) + b1_ref[...]
    z1_ref[...] = z1
    acc_ref[0:1, :] += jnp.sum(z1, axis=0, keepdims=True)
    acc_ref[1:2, :] += jnp.sum(z1 * z1, axis=0, keepdims=True)
    @pl.when(i == NBLK - 1)
    def _():
        st_ref[...] = acc_ref[...]


def _bn_coeff(st_ref, g_ref, bb_ref):
    m = st_ref[0:1, :] * (1.0 / N)
    v = st_ref[1:2, :] * (1.0 / N) - m * m
    a = g_ref[...] * lax.rsqrt(v + 1e-5)
    c = bb_ref[...] - m * a
    return a, c


def _kb_body(z1_ref, st_ref, g1_ref, bb1_ref, w2_ref, b2_ref,
             z2_ref, st2_ref, acc_ref):
    i = pl.program_id(0)
    @pl.when(i == 0)
    def _():
        acc_ref[...] = jnp.zeros_like(acc_ref)
    a, c = _bn_coeff(st_ref, g1_ref, bb1_ref)
    z1n = jnp.maximum(z1_ref[...] * a + c, 0.0)
    z2 = jnp.dot(z1n.astype(jnp.bfloat16), w2_ref[...].astype(jnp.bfloat16),
                 preferred_element_type=jnp.float32) + b2_ref[...]
    z2_ref[...] = z2
    acc_ref[0:1, :] += jnp.sum(z2, axis=0, keepdims=True)
    acc_ref[1:2, :] += jnp.sum(z2 * z2, axis=0, keepdims=True)
    @pl.when(i == NBLK - 1)
    def _():
        st2_ref[...] = acc_ref[...]


def _kc_body(z2_ref, st2_ref, g_ref, bb_ref, ec_ref, xe_ref):
    a, c = _bn_coeff(st2_ref, g_ref, bb_ref)
    h = jnp.maximum(z2_ref[...] * a + c, 0.0)
    _split_store(xe_ref, h + ec_ref[...])


def _kpool_body(z2_ref, st2_ref, g_ref, bb_ref, batch_ref, ow_ref, ob_ref,
                out_ref, pool_ref):
    i = pl.program_id(0)
    @pl.when(i == 0)
    def _():
        pool_ref[...] = jnp.zeros_like(pool_ref)
    a, c = _bn_coeff(st2_ref, g_ref, bb_ref)
    h = jnp.maximum(z2_ref[...] * a + c, 0.0)
    onehot = (batch_ref[0] == lax.broadcasted_iota(jnp.int32, (G, MB), 0)
              ).astype(jnp.float32)
    pool_ref[...] += jnp.dot(onehot, h, preferred_element_type=jnp.float32)
    @pl.when(i == NBLK - 1)
    def _():
        out_ref[...] = (jnp.dot(pool_ref[...], ow_ref[...],
                                preferred_element_type=jnp.float32)
                        + ob_ref[...])


def _full(shape):
    return pl.BlockSpec(shape, lambda i: tuple(0 for _ in shape))


def _rows(d):
    return pl.BlockSpec((MB, d), lambda i: (i, 0))


def _rows3():
    return pl.BlockSpec((XSL, MB, HSL), lambda i: (0, i, 0))


_SEQ = pltpu.CompilerParams(dimension_semantics=("arbitrary",))
_XE_SHAPE = jax.ShapeDtypeStruct((XSL, N, HSL), jnp.float32)


def _k_pre(x, node_W, node_b, P, W_aug):
    return pl.pallas_call(
        _pre_body,
        grid=(NBLK,),
        in_specs=[
            _rows(256),
            _full((256, H)),
            _full((1, H)),
            pl.BlockSpec((NC, MB, DEA), lambda i: (0, i, 0)),
            _full((DEA, H)),
        ],
        out_specs=[_rows3(), _rows(H)],
        out_shape=[_XE_SHAPE, jax.ShapeDtypeStruct((N, H), jnp.float32)],
        compiler_params=_SEQ,
    )(x, node_W, node_b, P, W_aug)


def _k_a(s, xe, agg, W1, b1):
    return pl.pallas_call(
        _ka_body,
        grid=(NBLK,),
        in_specs=[
            _full((1, 1)),
            _rows3(),
            pl.BlockSpec((XSL, MB, HSL), lambda i: (0, i, 0)),
            _full((H, 2 * H)),
            _full((1, 2 * H)),
        ],
        out_specs=[_rows(2 * H), _full((8, 2 * H))],
        out_shape=[jax.ShapeDtypeStruct((N, 2 * H), jnp.float32),
                   jax.ShapeDtypeStruct((8, 2 * H), jnp.float32)],
        scratch_shapes=[pltpu.VMEM((8, 2 * H), jnp.float32)],
        compiler_params=_SEQ,
    )(s, xe, agg, W1, b1)


def _k_b(z1, st1, g1, bb1, W2, b2):
    return pl.pallas_call(
        _kb_body,
        grid=(NBLK,),
        in_specs=[
            _rows(2 * H),
            _full((8, 2 * H)),
            _full((1, 2 * H)),
            _full((1, 2 * H)),
            _full((2 * H, H)),
            _full((1, H)),
        ],
        out_specs=[_rows(H), _full((8, H))],
        out_shape=[jax.ShapeDtypeStruct((N, H), jnp.float32),
                   jax.ShapeDtypeStruct((8, H), jnp.float32)],
        scratch_shapes=[pltpu.VMEM((8, H), jnp.float32)],
        compiler_params=_SEQ,
    )(z1, st1, g1, bb1, W2, b2)


def _k_c(z2, st2, g, bb, ec):
    return pl.pallas_call(
        _kc_body,
        grid=(NBLK,),
        in_specs=[
            _rows(H),
            _full((8, H)),
            _full((1, H)),
            _full((1, H)),
            _rows(H),
        ],
        out_specs=_rows3(),
        out_shape=_XE_SHAPE,
        compiler_params=_SEQ,
    )(z2, st2, g, bb, ec)


def _k_pool(z2, st2, g, bb, batch3, out_W, out_b):
    return pl.pallas_call(
        _kpool_body,
        grid=(NBLK,),
        in_specs=[
            _rows(H),
            _full((8, H)),
            _full((1, H)),
            _full((1, H)),
            pl.BlockSpec((1, 1, MB), lambda i: (i, 0, 0)),
            _full((H, 256)),
            _full((1, 256)),
        ],
        out_specs=_full((G, 256)),
        out_shape=jax.ShapeDtypeStruct((G, 256), jnp.float32),
        scratch_shapes=[pltpu.VMEM((G, H), jnp.float32)],
        compiler_params=_SEQ,
    )(z2, st2, g, bb, batch3, out_W, out_b)


def kernel(x, edge_index, edge_attr, batch, params):
    row = edge_index[0].astype(jnp.int32)
    col = edge_index[1].astype(jnp.int32)
    ea_aug = jnp.concatenate(
        [edge_attr,
         jnp.ones((E, 1), jnp.float32),
         jnp.zeros((E, DEA - 17), jnp.float32)], axis=1)
    W_aug = jnp.concatenate(
        [params['edge_W'],
         params['edge_b'][None, :],
         jnp.zeros((DEA - 17, H), jnp.float32)], axis=0)

    P = _sc_edge_feat(ea_aug, row, col)
    xe, ec = _k_pre(x, params['node_W'], params['node_b'][None, :], P, W_aug)

    out = None
    for l in range(L):
        p = params['layers'][l]
        agg = _sc_agg(xe, row, col)
        s = jnp.reshape(1.0 + p['eps'], (1, 1))
        z1, st1 = _k_a(s, xe, agg, p['W1'], p['b1'][None, :])
        z2, st2 = _k_b(z1, st1, p['g1'][None, :], p['bb1'][None, :],
                       p['W2'], p['b2'][None, :])
        if l < L - 1:
            xe = _k_c(z2, st2, p['g'][None, :], p['bb'][None, :], ec)
        else:
            batch3 = batch.astype(jnp.int32).reshape(NBLK, 1, MB)
            out = _k_pool(z2, st2, p['g'][None, :], p['bb'][None, :],
                          batch3, params['out_W'], params['out_b'][None, :])
    return out


# speedup vs baseline: 1.0002x; 1.0002x over previous
import functools

import jax
import jax.numpy as jnp
from jax import lax
from jax.experimental import pallas as pl
from jax.experimental.pallas import tpu as pltpu
from jax.experimental.pallas import tpu_sc as plsc

N = 10000
E = 160000
H = 512
G = 64
L = 3

NC = 2
NS = 16
NTILES = NC * NS

EPT = E // NS
XSL = 4
HSL = H // XSL
NPAD = 10240
RPT = NPAD // NS
KB = 80
NBATCH = EPT // KB

DEA = 128
EPT2 = E // NTILES
KB2 = 40
NB2 = EPT2 // KB2

MB = 1000
NBLK = N // MB

_mesh = plsc.VectorSubcoreMesh(core_axis_name="c", subcore_axis_name="s")


def _zero_vmem_2d(ref, rows, cols):
    z = jnp.zeros((16,), jnp.float32)
    for r in range(rows):
        def body(j, carry):
            ref[r, pl.ds(j * 16, 16)] = z
            return carry
        lax.fori_loop(0, cols // 16, body, 0)


@functools.partial(
    pl.kernel,
    out_type=jax.ShapeDtypeStruct((NC, NPAD, DEA), jnp.float32),
    mesh=_mesh,
    scratch_types=[
        pltpu.VMEM_SHARED((NPAD, DEA), jnp.float32),
        pltpu.VMEM((KB2,), jnp.int32),
        pltpu.VMEM((KB2,), jnp.int32),
        pltpu.VMEM((KB2, DEA), jnp.float32),
        pltpu.VMEM((KB2,), jnp.int32),
        pltpu.VMEM((KB2,), jnp.int32),
        pltpu.VMEM((KB2, DEA), jnp.float32),
        pltpu.VMEM((16, DEA), jnp.float32),
        pltpu.SemaphoreType.DMA,
        pltpu.SemaphoreType.DMA,
    ],
)
def _sc_edge_feat(ea_hbm, row_hbm, col_hbm, p_hbm,
                  acc, st_r0, st_c0, ea_buf0, st_r1, st_c1, ea_buf1, zbuf,
                  sem0, sem1):
    ci = lax.axis_index("c")
    si = lax.axis_index("s")
    w = ci * NS + si
    bufs = ((st_r0, st_c0, ea_buf0, sem0), (st_r1, st_c1, ea_buf1, sem1))
    _zero_vmem_2d(zbuf, 16, DEA)

    def zero_acc(k, carry):
        pltpu.sync_copy(zbuf, acc.at[pl.ds(si * RPT + k * 16, 16)])
        return carry
    lax.fori_loop(0, RPT // 16, zero_acc, 0)
    plsc.subcore_barrier()

    def srcs(j):
        e0 = w * EPT2 + j * KB2
        return (row_hbm.at[pl.ds(e0, KB2)], col_hbm.at[pl.ds(e0, KB2)],
                ea_hbm.at[pl.ds(e0, KB2)])

    def start(j, bset):
        sr, sc, se = srcs(j)
        pltpu.async_copy(sr, bset[0], bset[3])
        pltpu.async_copy(sc, bset[1], bset[3])
        pltpu.async_copy(se, bset[2], bset[3])

    def wait(j, bset):
        sr, sc, se = srcs(j)
        pltpu.make_async_copy(sr, bset[0], bset[3]).wait()
        pltpu.make_async_copy(sc, bset[1], bset[3]).wait()
        pltpu.make_async_copy(se, bset[2], bset[3]).wait()

    def scat(bset):
        pltpu.sync_copy(bset[2], acc.at[bset[0]], add=True)
        pltpu.sync_copy(bset[2], acc.at[bset[1]], add=True)

    start(0, bufs[0])

    def pair(g, carry):
        b0 = 2 * g
        start(b0 + 1, bufs[1])
        wait(b0, bufs[0])
        scat(bufs[0])
        start(b0 + 2, bufs[0])
        wait(b0 + 1, bufs[1])
        scat(bufs[1])
        return carry
    lax.fori_loop(0, (NB2 - 1) // 2, pair, 0)
    wait(NB2 - 1, bufs[0])
    scat(bufs[0])

    plsc.subcore_barrier()
    pltpu.sync_copy(acc.at[pl.ds(si * RPT, RPT)],
                    p_hbm.at[ci, pl.ds(si * RPT, RPT)])


@functools.partial(
    pl.kernel,
    out_type=jax.ShapeDtypeStruct((XSL, NPAD, HSL), jnp.float32),
    mesh=_mesh,
    scratch_types=[
        pltpu.VMEM_SHARED((NPAD, HSL), jnp.float32),
        pltpu.VMEM((EPT,), jnp.int32),
        pltpu.VMEM((EPT,), jnp.int32),
        pltpu.VMEM((KB,), jnp.int32),
        pltpu.VMEM((KB, HSL), jnp.float32),
        pltpu.VMEM((KB, HSL), jnp.float32),
        pltpu.VMEM((16, HSL), jnp.float32),
        pltpu.SemaphoreType.DMA,
        pltpu.SemaphoreType.DMA,
    ],
)
def _sc_agg(xe_hbm, row_hbm, col_hbm, agg_hbm,
            acc, row_v, col_v, stage, rows_buf0, rows_buf1, zbuf, sem0, sem1):
    ci = lax.axis_index("c")
    si = lax.axis_index("s")
    ebase = si * EPT
    pltpu.sync_copy(row_hbm.at[pl.ds(ebase, EPT)], row_v)
    pltpu.sync_copy(col_hbm.at[pl.ds(ebase, EPT)], col_v)
    _zero_vmem_2d(zbuf, 16, HSL)
    bufs = ((rows_buf0, sem0), (rows_buf1, sem1))

    for sl in range(XSL // NC):
        slidx = ci * (XSL // NC) + sl
        def zero_acc(k, carry):
            pltpu.sync_copy(zbuf, acc.at[pl.ds(si * RPT + k * 16, 16)])
            return carry
        lax.fori_loop(0, RPT // 16, zero_acc, 0)
        plsc.subcore_barrier()

        def gsrc(b):
            return xe_hbm.at[slidx].at[row_v.at[pl.ds(b * KB, KB)]]

        def start(b, bset):
            pltpu.async_copy(gsrc(b), bset[0], bset[1])

        def wait(b, bset):
            pltpu.make_async_copy(gsrc(b), bset[0], bset[1]).wait()

        def scat(b, bset):
            for k in range(KB // 16):
                stage[pl.ds(k * 16, 16)] = col_v[pl.ds(b * KB + k * 16, 16)]
            pltpu.sync_copy(bset[0], acc.at[stage], add=True)

        start(0, bufs[0])

        def pair(g, carry):
            b0 = 2 * g
            start(b0 + 1, bufs[1])
            wait(b0, bufs[0])
            scat(b0, bufs[0])
            start(b0 + 2, bufs[0])
            wait(b0 + 1, bufs[1])
            scat(b0 + 1, bufs[1])
            return carry
        lax.fori_loop(0, (NBATCH - 1) // 2, pair, 0)
        wait(NBATCH - 1, bufs[0])
        scat(NBATCH - 1, bufs[0])
        plsc.subcore_barrier()

        pltpu.sync_copy(acc.at[pl.ds(si * RPT, RPT)],
                        agg_hbm.at[slidx, pl.ds(si * RPT, RPT)])
        plsc.subcore_barrier()


def _cat(ref3):
    return jnp.concatenate([ref3[k] for k in range(XSL)], axis=1)


def _split_store(ref3, val):
    for k in range(XSL):
        ref3[k] = val[:, k * HSL:(k + 1) * HSL]


def _pre_body(x_ref, nw_ref, nb_ref, p_ref, wa_ref, xe_ref, ec_ref):
    h = jnp.maximum(
        jnp.dot(x_ref[...], nw_ref[...], preferred_element_type=jnp.float32)
        + nb_ref[...], 0.0)
    ec = jnp.dot(p_ref[0] + p_ref[1], wa_ref[...],
                 preferred_element_type=jnp.float32)
    ec_ref[...] = ec
    _split_store(xe_ref, h + ec)


def _ka_body(s_ref, xe_ref, agg_ref, w1_ref, b1_ref, z1_ref, st_ref, acc_ref):
    i = pl.program_id(0)
    @pl.when(i == 0)
    def _():
        acc_ref[...] = jnp.zeros_like(acc_ref)
    zpre = s_ref[0, 0] * _cat(xe_ref) + _cat(agg_ref)
    z1 = jnp.dot(zpre, w1_ref[...], preferred_element_type=jnp.float32) + b1_ref[...]
    z1_ref[...] = z1
    acc_ref[0:1, :] += jnp.sum(z1, axis=0, keepdims=True)
    acc_ref[1:2, :] += jnp.sum(z1 * z1, axis=0, keepdims=True)
    @pl.when(i == NBLK - 1)
    def _():
        st_ref[...] = acc_ref[...]


def _bn_coeff(st_ref, g_ref, bb_ref):
    m = st_ref[0:1, :] * (1.0 / N)
    v = st_ref[1:2, :] * (1.0 / N) - m * m
    a = g_ref[...] * lax.rsqrt(v + 1e-5)
    c = bb_ref[...] - m * a
    return a, c


def _kb_body(z1_ref, st_ref, g1_ref, bb1_ref, w2_ref, b2_ref,
             z2_ref, st2_ref, acc_ref):
    i = pl.program_id(0)
    @pl.when(i == 0)
    def _():
        acc_ref[...] = jnp.zeros_like(acc_ref)
    a, c = _bn_coeff(st_ref, g1_ref, bb1_ref)
    z1n = jnp.maximum(z1_ref[...] * a + c, 0.0)
    z2 = jnp.dot(z1n, w2_ref[...], preferred_element_type=jnp.float32) + b2_ref[...]
    z2_ref[...] = z2
    acc_ref[0:1, :] += jnp.sum(z2, axis=0, keepdims=True)
    acc_ref[1:2, :] += jnp.sum(z2 * z2, axis=0, keepdims=True)
    @pl.when(i == NBLK - 1)
    def _():
        st2_ref[...] = acc_ref[...]


def _kc_body(z2_ref, st2_ref, g_ref, bb_ref, ec_ref, xe_ref):
    a, c = _bn_coeff(st2_ref, g_ref, bb_ref)
    h = jnp.maximum(z2_ref[...] * a + c, 0.0)
    _split_store(xe_ref, h + ec_ref[...])


def _kpool_body(z2_ref, st2_ref, g_ref, bb_ref, batch_ref, ow_ref, ob_ref,
                out_ref, pool_ref):
    i = pl.program_id(0)
    @pl.when(i == 0)
    def _():
        pool_ref[...] = jnp.zeros_like(pool_ref)
    a, c = _bn_coeff(st2_ref, g_ref, bb_ref)
    h = jnp.maximum(z2_ref[...] * a + c, 0.0)
    onehot = (batch_ref[0] == lax.broadcasted_iota(jnp.int32, (G, MB), 0)
              ).astype(jnp.float32)
    pool_ref[...] += jnp.dot(onehot, h, preferred_element_type=jnp.float32)
    @pl.when(i == NBLK - 1)
    def _():
        out_ref[...] = (jnp.dot(pool_ref[...], ow_ref[...],
                                preferred_element_type=jnp.float32)
                        + ob_ref[...])


def _full(shape):
    return pl.BlockSpec(shape, lambda i: tuple(0 for _ in shape))


def _rows(d):
    return pl.BlockSpec((MB, d), lambda i: (i, 0))


def _rows3():
    return pl.BlockSpec((XSL, MB, HSL), lambda i: (0, i, 0))


_SEQ = pltpu.CompilerParams(dimension_semantics=("arbitrary",))
_XE_SHAPE = jax.ShapeDtypeStruct((XSL, N, HSL), jnp.float32)


def _k_pre(x, node_W, node_b, P, W_aug):
    return pl.pallas_call(
        _pre_body,
        grid=(NBLK,),
        in_specs=[
            _rows(256),
            _full((256, H)),
            _full((1, H)),
            pl.BlockSpec((NC, MB, DEA), lambda i: (0, i, 0)),
            _full((DEA, H)),
        ],
        out_specs=[_rows3(), _rows(H)],
        out_shape=[_XE_SHAPE, jax.ShapeDtypeStruct((N, H), jnp.float32)],
        compiler_params=_SEQ,
    )(x, node_W, node_b, P, W_aug)


def _k_a(s, xe, agg, W1, b1):
    return pl.pallas_call(
        _ka_body,
        grid=(NBLK,),
        in_specs=[
            _full((1, 1)),
            _rows3(),
            pl.BlockSpec((XSL, MB, HSL), lambda i: (0, i, 0)),
            _full((H, 2 * H)),
            _full((1, 2 * H)),
        ],
        out_specs=[_rows(2 * H), _full((8, 2 * H))],
        out_shape=[jax.ShapeDtypeStruct((N, 2 * H), jnp.float32),
                   jax.ShapeDtypeStruct((8, 2 * H), jnp.float32)],
        scratch_shapes=[pltpu.VMEM((8, 2 * H), jnp.float32)],
        compiler_params=_SEQ,
    )(s, xe, agg, W1, b1)


def _k_b(z1, st1, g1, bb1, W2, b2):
    return pl.pallas_call(
        _kb_body,
        grid=(NBLK,),
        in_specs=[
            _rows(2 * H),
            _full((8, 2 * H)),
            _full((1, 2 * H)),
            _full((1, 2 * H)),
            _full((2 * H, H)),
            _full((1, H)),
        ],
        out_specs=[_rows(H), _full((8, H))],
        out_shape=[jax.ShapeDtypeStruct((N, H), jnp.float32),
                   jax.ShapeDtypeStruct((8, H), jnp.float32)],
        scratch_shapes=[pltpu.VMEM((8, H), jnp.float32)],
        compiler_params=_SEQ,
    )(z1, st1, g1, bb1, W2, b2)


def _k_c(z2, st2, g, bb, ec):
    return pl.pallas_call(
        _kc_body,
        grid=(NBLK,),
        in_specs=[
            _rows(H),
            _full((8, H)),
            _full((1, H)),
            _full((1, H)),
            _rows(H),
        ],
        out_specs=_rows3(),
        out_shape=_XE_SHAPE,
        compiler_params=_SEQ,
    )(z2, st2, g, bb, ec)


def _k_pool(z2, st2, g, bb, batch3, out_W, out_b):
    return pl.pallas_call(
        _kpool_body,
        grid=(NBLK,),
        in_specs=[
            _rows(H),
            _full((8, H)),
            _full((1, H)),
            _full((1, H)),
            pl.BlockSpec((1, 1, MB), lambda i: (i, 0, 0)),
            _full((H, 256)),
            _full((1, 256)),
        ],
        out_specs=_full((G, 256)),
        out_shape=jax.ShapeDtypeStruct((G, 256), jnp.float32),
        scratch_shapes=[pltpu.VMEM((G, H), jnp.float32)],
        compiler_params=_SEQ,
    )(z2, st2, g, bb, batch3, out_W, out_b)


def kernel(x, edge_index, edge_attr, batch, params):
    row = edge_index[0].astype(jnp.int32)
    col = edge_index[1].astype(jnp.int32)
    ea_aug = jnp.concatenate(
        [edge_attr,
         jnp.ones((E, 1), jnp.float32),
         jnp.zeros((E, DEA - 17), jnp.float32)], axis=1)
    W_aug = jnp.concatenate(
        [params['edge_W'],
         params['edge_b'][None, :],
         jnp.zeros((DEA - 17, H), jnp.float32)], axis=0)

    P = _sc_edge_feat(ea_aug, row, col)
    xe, ec = _k_pre(x, params['node_W'], params['node_b'][None, :], P, W_aug)

    out = None
    for l in range(L):
        p = params['layers'][l]
        agg = _sc_agg(xe, row, col)
        s = jnp.reshape(1.0 + p['eps'], (1, 1))
        z1, st1 = _k_a(s, xe, agg, p['W1'], p['b1'][None, :])
        z2, st2 = _k_b(z1, st1, p['g1'][None, :], p['bb1'][None, :],
                       p['W2'], p['b2'][None, :])
        if l < L - 1:
            xe = _k_c(z2, st2, p['g'][None, :], p['bb'][None, :], ec)
        else:
            batch3 = batch.astype(jnp.int32).reshape(NBLK, 1, MB)
            out = _k_pool(z2, st2, p['g'][None, :], p['bb'][None, :],
                          batch3, params['out_W'], params['out_b'][None, :])
    return out


# speedup vs baseline: 1.0223x; 1.0222x over previous
import functools

import jax
import jax.numpy as jnp
from jax import lax
from jax.experimental import pallas as pl
from jax.experimental.pallas import tpu as pltpu
from jax.experimental.pallas import tpu_sc as plsc

N = 10000
E = 160000
H = 512
G = 64
L = 3

NC = 2
NS = 16
NTILES = NC * NS

EPT = E // NS
XSL = 4
HSL = H // XSL
NPAD = 10240
RPT = NPAD // NS
KB = 80
NBATCH = EPT // KB

DEA = 128
EPT2 = E // NTILES
KB2 = 40
NB2 = EPT2 // KB2

MB = 1000
NBLK = N // MB

_mesh = plsc.VectorSubcoreMesh(core_axis_name="c", subcore_axis_name="s")


def _zero_vmem_2d(ref, rows, cols):
    z = jnp.zeros((16,), jnp.float32)
    for r in range(rows):
        def body(j, carry):
            ref[r, pl.ds(j * 16, 16)] = z
            return carry
        lax.fori_loop(0, cols // 16, body, 0)


@functools.partial(
    pl.kernel,
    out_type=jax.ShapeDtypeStruct((NC, NPAD, DEA), jnp.float32),
    mesh=_mesh,
    scratch_types=[
        pltpu.VMEM_SHARED((NPAD, DEA), jnp.float32),
        pltpu.VMEM((KB2,), jnp.int32),
        pltpu.VMEM((KB2,), jnp.int32),
        pltpu.VMEM((KB2, DEA), jnp.float32),
        pltpu.VMEM((KB2,), jnp.int32),
        pltpu.VMEM((KB2,), jnp.int32),
        pltpu.VMEM((KB2, DEA), jnp.float32),
        pltpu.VMEM((16, DEA), jnp.float32),
        pltpu.SemaphoreType.DMA,
        pltpu.SemaphoreType.DMA,
    ],
)
def _sc_edge_feat(ea_hbm, row_hbm, col_hbm, p_hbm,
                  acc, st_r0, st_c0, ea_buf0, st_r1, st_c1, ea_buf1, zbuf,
                  sem0, sem1):
    ci = lax.axis_index("c")
    si = lax.axis_index("s")
    w = ci * NS + si
    bufs = ((st_r0, st_c0, ea_buf0, sem0), (st_r1, st_c1, ea_buf1, sem1))
    _zero_vmem_2d(zbuf, 16, DEA)

    def zero_acc(k, carry):
        pltpu.sync_copy(zbuf, acc.at[pl.ds(si * RPT + k * 16, 16)])
        return carry
    lax.fori_loop(0, RPT // 16, zero_acc, 0)
    plsc.subcore_barrier()

    def srcs(j):
        e0 = w * EPT2 + j * KB2
        return (row_hbm.at[pl.ds(e0, KB2)], col_hbm.at[pl.ds(e0, KB2)],
                ea_hbm.at[pl.ds(e0, KB2)])

    def start(j, bset):
        sr, sc, se = srcs(j)
        pltpu.async_copy(sr, bset[0], bset[3])
        pltpu.async_copy(sc, bset[1], bset[3])
        pltpu.async_copy(se, bset[2], bset[3])

    def wait(j, bset):
        sr, sc, se = srcs(j)
        pltpu.make_async_copy(sr, bset[0], bset[3]).wait()
        pltpu.make_async_copy(sc, bset[1], bset[3]).wait()
        pltpu.make_async_copy(se, bset[2], bset[3]).wait()

    def scat(bset):
        pltpu.sync_copy(bset[2], acc.at[bset[0]], add=True)
        pltpu.sync_copy(bset[2], acc.at[bset[1]], add=True)

    start(0, bufs[0])

    def pair(g, carry):
        b0 = 2 * g
        start(b0 + 1, bufs[1])
        wait(b0, bufs[0])
        scat(bufs[0])
        start(b0 + 2, bufs[0])
        wait(b0 + 1, bufs[1])
        scat(bufs[1])
        return carry
    lax.fori_loop(0, (NB2 - 1) // 2, pair, 0)
    wait(NB2 - 1, bufs[0])
    scat(bufs[0])

    plsc.subcore_barrier()
    pltpu.sync_copy(acc.at[pl.ds(si * RPT, RPT)],
                    p_hbm.at[ci, pl.ds(si * RPT, RPT)])


@functools.partial(
    pl.kernel,
    out_type=jax.ShapeDtypeStruct((XSL, NPAD, HSL), jnp.float32),
    mesh=_mesh,
    scratch_types=[
        pltpu.VMEM_SHARED((NPAD, HSL), jnp.float32),
        pltpu.VMEM((EPT,), jnp.int32),
        pltpu.VMEM((EPT,), jnp.int32),
        pltpu.VMEM((KB,), jnp.int32),
        pltpu.VMEM((KB, HSL), jnp.float32),
        pltpu.VMEM((KB, HSL), jnp.float32),
        pltpu.VMEM((16, HSL), jnp.float32),
        pltpu.SemaphoreType.DMA,
        pltpu.SemaphoreType.DMA,
    ],
)
def _sc_agg(xe_hbm, row_hbm, col_hbm, agg_hbm,
            acc, row_v, col_v, stage, rows_buf0, rows_buf1, zbuf, sem0, sem1):
    ci = lax.axis_index("c")
    si = lax.axis_index("s")
    ebase = si * EPT
    pltpu.sync_copy(row_hbm.at[pl.ds(ebase, EPT)], row_v)
    pltpu.sync_copy(col_hbm.at[pl.ds(ebase, EPT)], col_v)
    _zero_vmem_2d(zbuf, 16, HSL)
    bufs = ((rows_buf0, sem0), (rows_buf1, sem1))

    for sl in range(XSL // NC):
        slidx = ci * (XSL // NC) + sl
        def zero_acc(k, carry):
            pltpu.sync_copy(zbuf, acc.at[pl.ds(si * RPT + k * 16, 16)])
            return carry
        lax.fori_loop(0, RPT // 16, zero_acc, 0)
        plsc.subcore_barrier()

        def gsrc(b):
            return xe_hbm.at[slidx].at[row_v.at[pl.ds(b * KB, KB)]]

        def start(b, bset):
            pltpu.async_copy(gsrc(b), bset[0], bset[1])

        def wait(b, bset):
            pltpu.make_async_copy(gsrc(b), bset[0], bset[1]).wait()

        def scat(b, bset):
            for k in range(KB // 16):
                stage[pl.ds(k * 16, 16)] = col_v[pl.ds(b * KB + k * 16, 16)]
            pltpu.sync_copy(bset[0], acc.at[stage], add=True)

        start(0, bufs[0])

        def pair(g, carry):
            b0 = 2 * g
            start(b0 + 1, bufs[1])
            wait(b0, bufs[0])
            scat(b0, bufs[0])
            start(b0 + 2, bufs[0])
            wait(b0 + 1, bufs[1])
            scat(b0 + 1, bufs[1])
            return carry
        lax.fori_loop(0, (NBATCH - 1) // 2, pair, 0)
        wait(NBATCH - 1, bufs[0])
        scat(NBATCH - 1, bufs[0])
        plsc.subcore_barrier()

        pltpu.sync_copy(acc.at[pl.ds(si * RPT, RPT)],
                        agg_hbm.at[slidx, pl.ds(si * RPT, RPT)])
        plsc.subcore_barrier()


def _cat(ref3):
    return jnp.concatenate([ref3[k] for k in range(XSL)], axis=1)


def _split_store(ref3, val):
    for k in range(XSL):
        ref3[k] = val[:, k * HSL:(k + 1) * HSL]


def _pre_body(x_ref, nw_ref, nb_ref, p_ref, wa_ref, xe_ref, ec_ref):
    h = jnp.maximum(
        jnp.dot(x_ref[...], nw_ref[...], preferred_element_type=jnp.float32)
        + nb_ref[...], 0.0)
    ec = jnp.dot(p_ref[0] + p_ref[1], wa_ref[...],
                 preferred_element_type=jnp.float32)
    ec_ref[...] = ec
    _split_store(xe_ref, h + ec)


def _ka_body(s_ref, xe_ref, agg_ref, w1_ref, b1_ref, z1_ref, st_ref, acc_ref):
    i = pl.program_id(0)
    @pl.when(i == 0)
    def _():
        acc_ref[...] = jnp.zeros_like(acc_ref)
    zpre = s_ref[0, 0] * _cat(xe_ref) + _cat(agg_ref)
    z1 = jnp.dot(zpre, w1_ref[...], preferred_element_type=jnp.float32) + b1_ref[...]
    z1_ref[...] = z1.astype(jnp.bfloat16)
    acc_ref[0:1, :] += jnp.sum(z1, axis=0, keepdims=True)
    acc_ref[1:2, :] += jnp.sum(z1 * z1, axis=0, keepdims=True)
    @pl.when(i == NBLK - 1)
    def _():
        st_ref[...] = acc_ref[...]


def _bn_coeff(st_ref, g_ref, bb_ref):
    m = st_ref[0:1, :] * (1.0 / N)
    v = st_ref[1:2, :] * (1.0 / N) - m * m
    a = g_ref[...] * lax.rsqrt(v + 1e-5)
    c = bb_ref[...] - m * a
    return a, c


def _kb_body(z1_ref, st_ref, g1_ref, bb1_ref, w2_ref, b2_ref,
             z2_ref, st2_ref, acc_ref):
    i = pl.program_id(0)
    @pl.when(i == 0)
    def _():
        acc_ref[...] = jnp.zeros_like(acc_ref)
    a, c = _bn_coeff(st_ref, g1_ref, bb1_ref)
    z1n = jnp.maximum(z1_ref[...].astype(jnp.float32) * a + c, 0.0)
    z2 = jnp.dot(z1n, w2_ref[...], preferred_element_type=jnp.float32) + b2_ref[...]
    z2_ref[...] = z2
    acc_ref[0:1, :] += jnp.sum(z2, axis=0, keepdims=True)
    acc_ref[1:2, :] += jnp.sum(z2 * z2, axis=0, keepdims=True)
    @pl.when(i == NBLK - 1)
    def _():
        st2_ref[...] = acc_ref[...]


def _kc_body(z2_ref, st2_ref, g_ref, bb_ref, ec_ref, xe_ref):
    a, c = _bn_coeff(st2_ref, g_ref, bb_ref)
    h = jnp.maximum(z2_ref[...] * a + c, 0.0)
    _split_store(xe_ref, h + ec_ref[...])


def _kpool_body(z2_ref, st2_ref, g_ref, bb_ref, batch_ref, ow_ref, ob_ref,
                out_ref, pool_ref):
    i = pl.program_id(0)
    @pl.when(i == 0)
    def _():
        pool_ref[...] = jnp.zeros_like(pool_ref)
    a, c = _bn_coeff(st2_ref, g_ref, bb_ref)
    h = jnp.maximum(z2_ref[...] * a + c, 0.0)
    onehot = (batch_ref[0] == lax.broadcasted_iota(jnp.int32, (G, MB), 0)
              ).astype(jnp.float32)
    pool_ref[...] += jnp.dot(onehot, h, preferred_element_type=jnp.float32)
    @pl.when(i == NBLK - 1)
    def _():
        out_ref[...] = (jnp.dot(pool_ref[...], ow_ref[...],
                                preferred_element_type=jnp.float32)
                        + ob_ref[...])


def _full(shape):
    return pl.BlockSpec(shape, lambda i: tuple(0 for _ in shape))


def _rows(d):
    return pl.BlockSpec((MB, d), lambda i: (i, 0))


def _rows3():
    return pl.BlockSpec((XSL, MB, HSL), lambda i: (0, i, 0))


_SEQ = pltpu.CompilerParams(dimension_semantics=("arbitrary",))
_XE_SHAPE = jax.ShapeDtypeStruct((XSL, N, HSL), jnp.float32)


def _k_pre(x, node_W, node_b, P, W_aug):
    return pl.pallas_call(
        _pre_body,
        grid=(NBLK,),
        in_specs=[
            _rows(256),
            _full((256, H)),
            _full((1, H)),
            pl.BlockSpec((NC, MB, DEA), lambda i: (0, i, 0)),
            _full((DEA, H)),
        ],
        out_specs=[_rows3(), _rows(H)],
        out_shape=[_XE_SHAPE, jax.ShapeDtypeStruct((N, H), jnp.float32)],
        compiler_params=_SEQ,
    )(x, node_W, node_b, P, W_aug)


def _k_a(s, xe, agg, W1, b1):
    return pl.pallas_call(
        _ka_body,
        grid=(NBLK,),
        in_specs=[
            _full((1, 1)),
            _rows3(),
            pl.BlockSpec((XSL, MB, HSL), lambda i: (0, i, 0)),
            _full((H, 2 * H)),
            _full((1, 2 * H)),
        ],
        out_specs=[_rows(2 * H), _full((8, 2 * H))],
        out_shape=[jax.ShapeDtypeStruct((N, 2 * H), jnp.bfloat16),
                   jax.ShapeDtypeStruct((8, 2 * H), jnp.float32)],
        scratch_shapes=[pltpu.VMEM((8, 2 * H), jnp.float32)],
        compiler_params=_SEQ,
    )(s, xe, agg, W1, b1)


def _k_b(z1, st1, g1, bb1, W2, b2):
    return pl.pallas_call(
        _kb_body,
        grid=(NBLK,),
        in_specs=[
            _rows(2 * H),
            _full((8, 2 * H)),
            _full((1, 2 * H)),
            _full((1, 2 * H)),
            _full((2 * H, H)),
            _full((1, H)),
        ],
        out_specs=[_rows(H), _full((8, H))],
        out_shape=[jax.ShapeDtypeStruct((N, H), jnp.float32),
                   jax.ShapeDtypeStruct((8, H), jnp.float32)],
        scratch_shapes=[pltpu.VMEM((8, H), jnp.float32)],
        compiler_params=_SEQ,
    )(z1, st1, g1, bb1, W2, b2)


def _k_c(z2, st2, g, bb, ec):
    return pl.pallas_call(
        _kc_body,
        grid=(NBLK,),
        in_specs=[
            _rows(H),
            _full((8, H)),
            _full((1, H)),
            _full((1, H)),
            _rows(H),
        ],
        out_specs=_rows3(),
        out_shape=_XE_SHAPE,
        compiler_params=_SEQ,
    )(z2, st2, g, bb, ec)


def _k_pool(z2, st2, g, bb, batch3, out_W, out_b):
    return pl.pallas_call(
        _kpool_body,
        grid=(NBLK,),
        in_specs=[
            _rows(H),
            _full((8, H)),
            _full((1, H)),
            _full((1, H)),
            pl.BlockSpec((1, 1, MB), lambda i: (i, 0, 0)),
            _full((H, 256)),
            _full((1, 256)),
        ],
        out_specs=_full((G, 256)),
        out_shape=jax.ShapeDtypeStruct((G, 256), jnp.float32),
        scratch_shapes=[pltpu.VMEM((G, H), jnp.float32)],
        compiler_params=_SEQ,
    )(z2, st2, g, bb, batch3, out_W, out_b)


def kernel(x, edge_index, edge_attr, batch, params):
    row = edge_index[0].astype(jnp.int32)
    col = edge_index[1].astype(jnp.int32)
    ea_aug = jnp.concatenate(
        [edge_attr,
         jnp.ones((E, 1), jnp.float32),
         jnp.zeros((E, DEA - 17), jnp.float32)], axis=1)
    W_aug = jnp.concatenate(
        [params['edge_W'],
         params['edge_b'][None, :],
         jnp.zeros((DEA - 17, H), jnp.float32)], axis=0)

    P = _sc_edge_feat(ea_aug, row, col)
    xe, ec = _k_pre(x, params['node_W'], params['node_b'][None, :], P, W_aug)

    out = None
    for l in range(L):
        p = params['layers'][l]
        agg = _sc_agg(xe, row, col)
        s = jnp.reshape(1.0 + p['eps'], (1, 1))
        z1, st1 = _k_a(s, xe, agg, p['W1'], p['b1'][None, :])
        z2, st2 = _k_b(z1, st1, p['g1'][None, :], p['bb1'][None, :],
                       p['W2'], p['b2'][None, :])
        if l < L - 1:
            xe = _k_c(z2, st2, p['g'][None, :], p['bb'][None, :], ec)
        else:
            batch3 = batch.astype(jnp.int32).reshape(NBLK, 1, MB)
            out = _k_pool(z2, st2, p['g'][None, :], p['bb'][None, :],
                          batch3, params['out_W'], params['out_b'][None, :])
    return out
